# baseline (speedup 1.0000x reference)
import functools

import jax
import jax.numpy as jnp
from jax import lax
from jax.experimental import pallas as pl
from jax.experimental.pallas import tpu as pltpu
from jax.experimental.pallas import tpu_sc as plsc

L = 50
E = 128
H = 256
NCLS = 10
NLANE = 16
NVEC = E // NLANE
SEG_PER_STREAM = 4
ROWS_PER_STREAM = SEG_PER_STREAM * L
NBUF = 4
TABLE_ROWS = (0, 2, 3)


def _sc_pool(emb_word, emb_bi, emb_tri, idx_flat, batch):
    info = plsc.get_sparse_core_info()
    nw = info.num_cores * info.num_subcores
    nc = info.num_cores
    rpw = batch // nw
    n_stream = rpw // SEG_PER_STREAM
    n_chunk = n_stream // NBUF
    mesh = plsc.VectorSubcoreMesh(core_axis_name="c", subcore_axis_name="s")

    @functools.partial(
        pl.kernel,
        mesh=mesh,
        out_type=jax.ShapeDtypeStruct((3, batch, E), jnp.float32),
        scratch_types=[
            pltpu.VMEM((rpw * L,), jnp.int32),
            [pltpu.VMEM((ROWS_PER_STREAM, E), jnp.float32)
             for _ in range(NBUF)],
            pltpu.VMEM((rpw, E), jnp.float32),
            [pltpu.SemaphoreType.DMA for _ in range(NBUF)],
        ],
    )
    def pool_kernel(word_ref, bi_ref, tri_ref, idx_ref, out_ref,
                    idx_v, rows_bufs, pool_v, sems):
        wid = lax.axis_index("s") * nc + lax.axis_index("c")
        base = wid * rpw

        def fire(table, g, b):
            off = pl.multiple_of(g * ROWS_PER_STREAM, 8)
            pltpu.async_copy(
                table.at[idx_v.at[pl.ds(off, ROWS_PER_STREAM)]],
                rows_bufs[b], sems[b])

        def drain(table, b):
            pltpu.make_async_copy(
                table.at[idx_v.at[pl.ds(0, ROWS_PER_STREAM)]],
                rows_bufs[b], sems[b]).wait()

        def reduce(g, b):
            for s in range(SEG_PER_STREAM):
                def red_body(l, accs, s=s):
                    row = s * L + 2 * l
                    return tuple(
                        accs[j]
                        + rows_bufs[b][row, pl.ds(j * NLANE, NLANE)]
                        + rows_bufs[b][row + 1, pl.ds(j * NLANE, NLANE)]
                        for j in range(NVEC))
                accs = lax.fori_loop(
                    0, L // 2, red_body,
                    tuple(jnp.zeros((NLANE,), jnp.float32)
                          for _ in range(NVEC)))
                seg = g * SEG_PER_STREAM + s
                for j in range(NVEC):
                    pool_v[seg, pl.ds(j * NLANE, NLANE)] = accs[j] * (1.0 / L)

        for t, table in enumerate((word_ref, bi_ref, tri_ref)):
            pltpu.sync_copy(
                idx_ref.at[pl.ds(TABLE_ROWS[t] * batch * L + base * L,
                                 rpw * L)], idx_v)
            for b in range(NBUF):
                fire(table, b, b)

            def chunk_body(c, carry, table=table):
                for b in range(NBUF):
                    drain(table, b)
                    reduce(c * NBUF + b, b)

                    @pl.when(c < n_chunk - 1)
                    def _():
                        fire(table, c * NBUF + b + NBUF, b)
                return carry

            lax.fori_loop(0, n_chunk, chunk_body, 0)
            pltpu.sync_copy(pool_v, out_ref.at[t, pl.ds(base, rpw), :])

    return pool_kernel(emb_word, emb_bi, emb_tri, idx_flat)


def _mlp(pooled, W1, b1, W2, b2, batch):
    w1r = W1.reshape(3, E, H)
    b1r = b1.reshape(1, H)
    w2p = jnp.pad(W2, ((0, 0), (0, 128 - NCLS)))
    b2p = jnp.pad(b2, (0, 128 - NCLS)).reshape(1, 128)
    bb = 512

    def body(p_ref, w1_ref, b1_ref, w2_ref, b2_ref, o_ref):
        p = p_ref[...]
        w1 = w1_ref[...]
        h = (jnp.dot(p[0], w1[0], preferred_element_type=jnp.float32)
             + jnp.dot(p[1], w1[1], preferred_element_type=jnp.float32)
             + jnp.dot(p[2], w1[2], preferred_element_type=jnp.float32)
             + b1_ref[...])
        h = jnp.maximum(h, 0.0)
        o_ref[...] = (jnp.dot(h, w2_ref[...],
                              preferred_element_type=jnp.float32)
                      + b2_ref[...])

    out = pl.pallas_call(
        body,
        grid=(batch // bb,),
        in_specs=[
            pl.BlockSpec((3, bb, E), lambda i: (0, i, 0)),
            pl.BlockSpec((3, E, H), lambda i: (0, 0, 0)),
            pl.BlockSpec((1, H), lambda i: (0, 0)),
            pl.BlockSpec((H, 128), lambda i: (0, 0)),
            pl.BlockSpec((1, 128), lambda i: (0, 0)),
        ],
        out_specs=pl.BlockSpec((bb, 128), lambda i: (i, 0)),
        out_shape=jax.ShapeDtypeStruct((batch, 128), jnp.float32),
    )(pooled, w1r, b1r, w2p, b2p)
    return out[:, :NCLS]


def kernel(x, emb_word, emb_bi, emb_tri, W1, b1, W2, b2):
    batch = x.shape[1]
    idx = x.astype(jnp.int32).reshape(4 * batch * L)
    pooled = _sc_pool(emb_word, emb_bi, emb_tri, idx, batch)
    return _mlp(pooled, W1, b1, W2, b2, batch)


# speedup vs baseline: 1.0658x; 1.0658x over previous
import functools

import jax
import jax.numpy as jnp
from jax import lax
from jax.experimental import pallas as pl
from jax.experimental.pallas import tpu as pltpu
from jax.experimental.pallas import tpu_sc as plsc

L = 50
E = 128
H = 256
NCLS = 10
NLANE = 16
NVEC = E // NLANE
SEG_PER_STREAM = 4
ROWS_PER_STREAM = SEG_PER_STREAM * L
NBUF = 4
TABLE_ROWS = (0, 2, 3)


def _sc_pool(emb_word, emb_bi, emb_tri, idx_flat, batch):
    info = plsc.get_sparse_core_info()
    nw = info.num_cores * info.num_subcores
    nc = info.num_cores
    rpw = batch // nw
    n_stream = rpw // SEG_PER_STREAM
    n_chunk = n_stream // NBUF
    mesh = plsc.VectorSubcoreMesh(core_axis_name="c", subcore_axis_name="s")

    segs_per_chunk = NBUF * SEG_PER_STREAM

    @functools.partial(
        pl.kernel,
        mesh=mesh,
        out_type=jax.ShapeDtypeStruct((3, batch, E), jnp.float32),
        scratch_types=[
            [pltpu.VMEM((rpw * L,), jnp.int32) for _ in range(2)],
            [pltpu.VMEM((ROWS_PER_STREAM, E), jnp.float32)
             for _ in range(NBUF)],
            pltpu.VMEM((segs_per_chunk, E), jnp.float32),
            [pltpu.SemaphoreType.DMA for _ in range(NBUF)],
        ],
    )
    def pool_kernel(word_ref, bi_ref, tri_ref, idx_ref, out_ref,
                    idx_bufs, rows_bufs, pool_v, sems):
        wid = lax.axis_index("s") * nc + lax.axis_index("c")
        base = wid * rpw
        tables = (word_ref, bi_ref, tri_ref)

        def stage_idx(t, ib):
            pltpu.sync_copy(
                idx_ref.at[pl.ds(TABLE_ROWS[t] * batch * L + base * L,
                                 rpw * L)], idx_bufs[ib])

        def fire(table, ib, g, b):
            off = pl.multiple_of(g * ROWS_PER_STREAM, 8)
            pltpu.async_copy(
                table.at[idx_bufs[ib].at[pl.ds(off, ROWS_PER_STREAM)]],
                rows_bufs[b], sems[b])

        def drain(table, b):
            pltpu.make_async_copy(
                table.at[idx_bufs[0].at[pl.ds(0, ROWS_PER_STREAM)]],
                rows_bufs[b], sems[b]).wait()

        def reduce(b):
            for s in range(SEG_PER_STREAM):
                def red_body(l, accs, s=s):
                    row = s * L + 2 * l
                    return tuple(
                        accs[j]
                        + rows_bufs[b][row, pl.ds(j * NLANE, NLANE)]
                        + rows_bufs[b][row + 1, pl.ds(j * NLANE, NLANE)]
                        for j in range(NVEC))
                accs = lax.fori_loop(
                    0, L // 2, red_body,
                    tuple(jnp.zeros((NLANE,), jnp.float32)
                          for _ in range(NVEC)))
                for j in range(NVEC):
                    pool_v[b * SEG_PER_STREAM + s,
                           pl.ds(j * NLANE, NLANE)] = accs[j] * (1.0 / L)

        stage_idx(0, 0)
        for b in range(NBUF):
            fire(tables[0], 0, b, b)
        for t in range(3):
            cur, nxt = t % 2, (t + 1) % 2
            if t < 2:
                stage_idx(t + 1, nxt)

            def chunk_body(c, carry, t=t, cur=cur, nxt=nxt):
                for b in range(NBUF):
                    drain(tables[t], b)
                    reduce(b)

                    @pl.when(c < n_chunk - 1)
                    def _():
                        fire(tables[t], cur, c * NBUF + b + NBUF, b)
                    if t < 2:
                        @pl.when(c == n_chunk - 1)
                        def _():
                            fire(tables[t + 1], nxt, b, b)
                pltpu.sync_copy(
                    pool_v,
                    out_ref.at[t, pl.ds(base + c * segs_per_chunk,
                                        segs_per_chunk), :])
                return carry

            lax.fori_loop(0, n_chunk, chunk_body, 0)

    return pool_kernel(emb_word, emb_bi, emb_tri, idx_flat)


def _mlp(pooled, W1, b1, W2, b2, batch):
    w1r = W1.reshape(3, E, H)
    b1r = b1.reshape(1, H)
    w2p = jnp.pad(W2, ((0, 0), (0, 128 - NCLS)))
    b2p = jnp.pad(b2, (0, 128 - NCLS)).reshape(1, 128)
    bb = 512

    def body(p_ref, w1_ref, b1_ref, w2_ref, b2_ref, o_ref):
        p = p_ref[...]
        w1 = w1_ref[...]
        h = (jnp.dot(p[0], w1[0], preferred_element_type=jnp.float32)
             + jnp.dot(p[1], w1[1], preferred_element_type=jnp.float32)
             + jnp.dot(p[2], w1[2], preferred_element_type=jnp.float32)
             + b1_ref[...])
        h = jnp.maximum(h, 0.0)
        o_ref[...] = (jnp.dot(h, w2_ref[...],
                              preferred_element_type=jnp.float32)
                      + b2_ref[...])

    out = pl.pallas_call(
        body,
        grid=(batch // bb,),
        in_specs=[
            pl.BlockSpec((3, bb, E), lambda i: (0, i, 0)),
            pl.BlockSpec((3, E, H), lambda i: (0, 0, 0)),
            pl.BlockSpec((1, H), lambda i: (0, 0)),
            pl.BlockSpec((H, 128), lambda i: (0, 0)),
            pl.BlockSpec((1, 128), lambda i: (0, 0)),
        ],
        out_specs=pl.BlockSpec((bb, 128), lambda i: (i, 0)),
        out_shape=jax.ShapeDtypeStruct((batch, 128), jnp.float32),
    )(pooled, w1r, b1r, w2p, b2p)
    return out[:, :NCLS]


def kernel(x, emb_word, emb_bi, emb_tri, W1, b1, W2, b2):
    batch = x.shape[1]
    idx = x.astype(jnp.int32).reshape(4 * batch * L)
    pooled = _sc_pool(emb_word, emb_bi, emb_tri, idx, batch)
    return _mlp(pooled, W1, b1, W2, b2, batch)
